# baseline (speedup 1.0000x reference)
import jax
import jax.numpy as jnp
from jax import lax
from jax.experimental import pallas as pl
from jax.experimental.pallas import tpu as pltpu
from jax.experimental.pallas import tpu_sc as plsc

N = 100000
K = 64
KH = 16
NP = 100096
NB = 102
NBPAD = 112
R_MIN = 0.0
R_MAX = 10.0


_BP = 5888
_NBLK = NP // _BP


def _tc_bins_body(x_ref, o_ref):
    i = pl.program_id(0)
    v = x_ref[...]
    sq = v * v
    s = sq[:, 0, :] + sq[:, 1, :] + sq[:, 2, :]
    r = jnp.sqrt(s)
    t = (r / 10.0) * 102
    t = jnp.minimum(t, 101.5)
    b = t.astype(jnp.int32)
    col = i * _BP + lax.broadcasted_iota(jnp.int32, (KH, _BP), 1)
    b = jnp.where(col < N, b, NB - 1)
    o_ref[...] = b[:, : _BP // 2] | (b[:, _BP // 2:] << 16)


def _tc_bins(t, half):
    return pl.pallas_call(
        _tc_bins_body,
        grid=(_NBLK,),
        in_specs=[pl.BlockSpec((KH, 4, _BP), lambda i, h=half: (h, 0, i))],
        out_specs=pl.BlockSpec((KH, _BP // 2), lambda i: (0, i)),
        out_shape=jax.ShapeDtypeStruct((KH, NP // 2), jnp.int32),
    )(t)


_NC = 2
_NS = 16
_NW = _NC * _NS
_ROWW = NP // 2
_CHUNK = 2944
_NCHUNK = _ROWW // _CHUNK


def _sc_hist_body(b_hbm, out_hbm, buf, hist):
    wid = lax.axis_index("s") * _NC + lax.axis_index("c")
    lanes = lax.iota(jnp.int32, 16)
    lane_base = lanes * NBPAD
    hi_base = lane_base + 16 * NBPAD
    ones = jnp.ones((16,), jnp.float32)
    zeros = jnp.zeros((16,), jnp.float32)
    for j in range(2 * 16 * NBPAD // 16):
        hist[pl.ds(j * 16, 16)] = zeros
    row = wid // 2
    half = wid % 2
    col0 = half * (9 * _CHUNK)

    def do_chunk(c):
        pltpu.sync_copy(b_hbm.at[row, pl.ds(col0 + c * _CHUNK, _CHUNK)], buf)

        def body(i, carry):
            base = i * 64
            for u in range(4):
                w = buf[pl.ds(base + u * 16, 16)]
                lo = w & 0xFFFF
                hi = lax.shift_right_logical(w, 16)
                plsc.addupdate_scatter(hist, [lane_base + lo], ones)
                plsc.addupdate_scatter(hist, [hi_base + hi], ones)
            return carry

        lax.fori_loop(0, _CHUNK // 64, body, 0)

    for c in range(8):
        do_chunk(c)

    @pl.when(half == 0)
    def _():
        do_chunk(8)

    pltpu.sync_copy(hist, out_hbm.at[wid])


def _sc_hist(b_words):
    mesh = plsc.VectorSubcoreMesh(core_axis_name="c", subcore_axis_name="s")
    f = pl.kernel(
        _sc_hist_body,
        mesh=mesh,
        out_type=jax.ShapeDtypeStruct((_NW, 2 * 16 * NBPAD), jnp.float32),
        scratch_types=[
            pltpu.VMEM((_CHUNK,), jnp.int32),
            pltpu.VMEM((2 * 16 * NBPAD,), jnp.float32),
        ],
        compiler_params=pltpu.CompilerParams(needs_layout_passes=False),
    )
    return f(b_words)


def kernel(nlist, positions):
    t = jnp.transpose(nlist, (1, 2, 0))
    parts = []
    for q in range(K // KH):
        bins = _tc_bins(t, q)
        parts.append(_sc_hist(bins))
    acc = parts[0]
    for p in parts[1:]:
        acc = acc + p
    hist = acc.reshape(_NW * 32, NBPAD).sum(0)
    shell_rs = jnp.linspace(R_MIN, R_MAX, 101)
    vols = shell_rs[1:] ** 3 - shell_rs[:-1] ** 3
    return hist[1:NB - 1] / vols


# speedup vs baseline: 1.1499x; 1.1499x over previous
import jax
import jax.numpy as jnp
from jax import lax
from jax.experimental import pallas as pl
from jax.experimental.pallas import tpu as pltpu
from jax.experimental.pallas import tpu_sc as plsc

N = 100000
K = 64
KH = 32
NP = 100096
NB = 102
NBPAD = 112
R_MIN = 0.0
R_MAX = 10.0


_BP = 5888
_NBLK = NP // _BP


def _tc_bins_body(x_ref, o_ref):
    i = pl.program_id(0)
    v = x_ref[...]
    sq = v * v
    s = sq[:, 0, :] + sq[:, 1, :] + sq[:, 2, :]
    r = jnp.sqrt(s)
    t = (r / 10.0) * 102
    t = jnp.minimum(t, 101.5)
    b = t.astype(jnp.int32)
    col = i * _BP + lax.broadcasted_iota(jnp.int32, (KH, _BP), 1)
    b = jnp.where(col < N, b, NB - 1)
    o_ref[...] = b[:, : _BP // 2] | (b[:, _BP // 2:] << 16)


def _tc_bins(t, half):
    return pl.pallas_call(
        _tc_bins_body,
        grid=(_NBLK,),
        in_specs=[pl.BlockSpec((KH, 4, _BP), lambda i, h=half: (h, 0, i))],
        out_specs=pl.BlockSpec((KH, _BP // 2), lambda i: (0, i)),
        out_shape=jax.ShapeDtypeStruct((KH, NP // 2), jnp.int32),
    )(t)


_NC = 2
_NS = 16
_NW = _NC * _NS
_ROWW = NP // 2
_CHUNK = 2944
_NCHUNK = _ROWW // _CHUNK


def _sc_hist_body(b_hbm, out_hbm, buf0, buf1, hist, sem0, sem1):
    wid = lax.axis_index("s") * _NC + lax.axis_index("c")
    lanes = lax.iota(jnp.int32, 16)
    lane_base = lanes * NBPAD
    hi_base = lane_base + 16 * NBPAD
    ones = jnp.ones((16,), jnp.float32)
    zeros = jnp.zeros((16,), jnp.float32)
    for j in range(2 * 16 * NBPAD // 16):
        hist[pl.ds(j * 16, 16)] = zeros
    bufs = (buf0, buf1)
    sems = (sem0, sem1)

    def start(k):
        return pltpu.async_copy(
            b_hbm.at[wid, pl.ds(k * _CHUNK, _CHUNK)], bufs[k % 2], sems[k % 2])

    pending = start(0)
    for k in range(_NCHUNK):
        nxt = start(k + 1) if k + 1 < _NCHUNK else None
        pending.wait()
        buf = bufs[k % 2]

        def body(i, carry):
            base = i * 64
            for u in range(4):
                w = buf[pl.ds(base + u * 16, 16)]
                lo = w & 0xFFFF
                hi = lax.shift_right_logical(w, 16)
                plsc.addupdate_scatter(hist, [lane_base + lo], ones)
                plsc.addupdate_scatter(hist, [hi_base + hi], ones)
            return carry

        lax.fori_loop(0, _CHUNK // 64, body, 0)
        pending = nxt
    pltpu.sync_copy(hist, out_hbm.at[wid])


def _sc_hist(b_words):
    mesh = plsc.VectorSubcoreMesh(core_axis_name="c", subcore_axis_name="s")
    f = pl.kernel(
        _sc_hist_body,
        mesh=mesh,
        out_type=jax.ShapeDtypeStruct((_NW, 2 * 16 * NBPAD), jnp.float32),
        scratch_types=[
            pltpu.VMEM((_CHUNK,), jnp.int32),
            pltpu.VMEM((_CHUNK,), jnp.int32),
            pltpu.VMEM((2 * 16 * NBPAD,), jnp.float32),
            pltpu.SemaphoreType.DMA,
            pltpu.SemaphoreType.DMA,
        ],
        compiler_params=pltpu.CompilerParams(needs_layout_passes=False),
    )
    return f(b_words)


def kernel(nlist, positions):
    t = jnp.transpose(nlist, (1, 2, 0))
    bins0 = _tc_bins(t, 0)
    part0 = _sc_hist(bins0)
    bins1 = _tc_bins(t, 1)
    part1 = _sc_hist(bins1)
    hist = (part0.reshape(_NW * 32, NBPAD) + part1.reshape(_NW * 32, NBPAD)
            ).sum(0)
    shell_rs = jnp.linspace(R_MIN, R_MAX, 101)
    vols = shell_rs[1:] ** 3 - shell_rs[:-1] ** 3
    return hist[1:NB - 1] / vols
